# baseline (speedup 1.0000x reference)
import functools

import jax
import jax.numpy as jnp
from jax import lax
from jax.experimental import pallas as pl
from jax.experimental.pallas import tpu as pltpu
from jax.experimental.pallas import tpu_sc as plsc

_EPS = 1e-6


def _make_sc_kernel(B, V, D, n_cores, n_subcores):
    nw = n_cores * n_subcores
    b_per_w = B // nw
    mesh = plsc.VectorSubcoreMesh(core_axis_name="c", subcore_axis_name="s")

    @functools.partial(
        pl.kernel,
        mesh=mesh,
        compiler_params=pltpu.CompilerParams(use_tc_tiling_on_sc=False),
        out_type=[
            jax.ShapeDtypeStruct((B, D), jnp.float32),
            jax.ShapeDtypeStruct((B, D), jnp.float32),
        ],
        scratch_types=[
            pltpu.VMEM((b_per_w,), jnp.int32),
            pltpu.VMEM((b_per_w, D), jnp.float32),
            pltpu.VMEM((b_per_w, D), jnp.float32),
            pltpu.VMEM((b_per_w, D), jnp.float32),
            pltpu.SemaphoreType.DMA,
        ],
    )
    def sc_kernel(idx_hbm, table_hbm, pz_hbm, z_hbm, idx_v, rows_v, pz_v, z_v, sem):
        wid = lax.axis_index("s") * n_cores + lax.axis_index("c")
        base = wid * b_per_w
        pltpu.sync_copy(idx_hbm.at[pl.ds(base, b_per_w)], idx_v)
        pltpu.async_copy(table_hbm.at[idx_v], rows_v, sem).wait()

        def body(i, carry):
            x = rows_v[i, :]
            p = 1.0 / (1.0 + jnp.exp(-x))
            p = p * (1.0 - 2.0 * _EPS) + _EPS
            pz_v[i, :] = p
            z_v[i, :] = jnp.where(p > 0.5, 1.0, 0.0)
            return carry

        lax.fori_loop(0, b_per_w, body, 0)

        pltpu.sync_copy(pz_v, pz_hbm.at[pl.ds(base, b_per_w)])
        pltpu.sync_copy(z_v, z_hbm.at[pl.ds(base, b_per_w)])

    return sc_kernel


def kernel(inputs, embedding):
    B = inputs.shape[0]
    V, D = embedding.shape
    info = plsc.get_sparse_core_info()
    idx = inputs.reshape(-1).astype(jnp.int32)
    sc = _make_sc_kernel(B, V, D, info.num_cores, info.num_subcores)
    pz, z_f = sc(idx, embedding)
    return pz, z_f.astype(jnp.bool_)
